# baseline (speedup 1.0000x reference)
import dataclasses
import functools

import jax
import jax.numpy as jnp
from jax import lax
from jax.experimental import pallas as pl
from jax.experimental.pallas import tpu as pltpu
from jax.experimental.pallas import tpu_sc as plsc

EMBED_DIM = 768
TOPK = 8
NUM_SAMPLES = 500
SIGMA = 0.05
B, N = 4, 2048
C = EMBED_DIM // 2

NLANES = 16
NWORKERS = 32
SLOTS = NWORKERS // B
GROUPS_PER_TEC = 4
CAP = 256
NROWS = B * NUM_SAMPLES
NGROUPS = NROWS // NLANES

_INT_MIN = jnp.iinfo(jnp.int32).min
_INV_SQRT2 = 0.7071067811865476


def _gelu_exact(v):
    return v * 0.5 * (1.0 + jax.lax.erf(v * _INV_SQRT2))


def _scores_kernel(x_ref, ids_ref, amn_ref, lng_ref, lnb_ref,
                   w1_ref, b1_ref, w2a_ref, w2b_ref, w3_ref,
                   score_ref, cls_ref):
    xb = x_ref[0]
    mu = jnp.mean(xb, axis=-1, keepdims=True)
    xc = xb - mu
    var = jnp.mean(xc * xc, axis=-1, keepdims=True)
    ln = xc / jnp.sqrt(var + 1e-5) * lng_ref[0][None, :] + lnb_ref[0][None, :]
    h = _gelu_exact(
        jnp.dot(ln, w1_ref[...], preferred_element_type=jnp.float32)
        + b1_ref[0][None, :])

    ids = ids_ref[0]
    iota = jax.lax.broadcasted_iota(jnp.int32, (1, N), 1)
    ikey = ids * N + (N - 1 - iota)
    gmax = jnp.max(ikey)
    gsel = (ikey == gmax).astype(jnp.float32)
    h_g = jnp.dot(gsel, h, preferred_element_type=jnp.float32)
    cls_ref[0] = jnp.dot(gsel, xb, preferred_element_type=jnp.float32)

    bias2 = jnp.dot(h_g, w2b_ref[...], preferred_element_type=jnp.float32)
    o = _gelu_exact(
        jnp.dot(h, w2a_ref[...], preferred_element_type=jnp.float32) + bias2)
    s = jax.nn.sigmoid(
        jnp.dot(o, w3_ref[...], preferred_element_type=jnp.float32))
    score_ref[0] = s.reshape(1, N) * amn_ref[0]


def _pack_kernel(noise_ref, score_ref, keys_ref):
    iota = jax.lax.broadcasted_iota(jnp.int32, (1, N), 1)
    p = score_ref[0] + noise_ref[0] * SIGMA
    q = ((p + 0.5) * 262144.0).astype(jnp.int32)
    keys_ref[0] = q * N + (N - 1 - iota)


def _sc_topk_kernel(keys_hbm, out_hbm, tile_ref, cand_ref, cnts_ref):
    lanes = lax.iota(jnp.int32, NLANES)
    zero16f = jnp.zeros((NLANES,), jnp.float32)
    wid = lax.axis_index("subcore") * 2 + lax.axis_index("core")

    for r in range(B * TOPK):
        @pl.loop(0, N, step=NLANES)
        def _(cc):
            cnts_ref[r, pl.ds(cc, NLANES)] = zero16f

    for t in range(GROUPS_PER_TEC):
        g = wid + t * NWORKERS

        @pl.when(g < NGROUPS)
        def _():
            gbase = NLANES * g
            pltpu.sync_copy(keys_hbm.at[pl.ds(gbase, NLANES)], tile_ref)

            def p1_body(i, ms):
                out = []
                for u in range(8):
                    j = i * 8 + u
                    col = jnp.full((NLANES,), j, jnp.int32)
                    v = plsc.load_gather(tile_ref, [lanes, col])
                    out.append(jnp.maximum(ms[u], v))
                return tuple(out)

            minit = tuple(jnp.full((NLANES,), _INT_MIN, jnp.int32)
                          for _ in range(8))
            ms = lax.fori_loop(0, N // 8, p1_body, minit)
            thr = ms[0]
            for u in range(1, 8):
                thr = jnp.minimum(thr, ms[u])

            def p2_body(i, cnt):
                for u in range(4):
                    j = i * 4 + u
                    col = jnp.full((NLANES,), j, jnp.int32)
                    v = plsc.load_gather(tile_ref, [lanes, col])
                    msk = v >= thr
                    cols = jnp.minimum(cnt, CAP - 1)
                    plsc.store_scatter(cand_ref, [lanes, cols], v, mask=msk)
                    cnt = cnt + msk.astype(jnp.int32)
                return cnt

            cnt = lax.fori_loop(0, N // 4, p2_body,
                                jnp.zeros((NLANES,), jnp.int32))

            for l in range(NLANES):
                cl = jnp.max(jnp.where(lanes == l, cnt, 0))
                nv = (cl + NLANES - 1) // NLANES
                bl = (gbase + l) // NUM_SAMPLES

                def merge(vi, best):
                    vec = cand_ref[l, pl.ds(vi * NLANES, NLANES)]
                    valid = (vi * NLANES + lanes) < cl
                    vec = jnp.where(valid, vec, lanes)
                    asc = lax.sort(vec)
                    m = jnp.maximum(best, asc)
                    best, _unused = plsc.sort_key_val(m, m, descending=True)
                    return best

                best = lax.fori_loop(0, nv, merge, 15 - lanes)
                idx = (N - 1) - (best & (N - 1))
                svals = jnp.where(lanes < TOPK, idx, 4064 + lanes)
                sidx = lax.sort(svals)
                plsc.addupdate_scatter(
                    cnts_ref,
                    [bl * TOPK + jnp.minimum(lanes, TOPK - 1),
                     jnp.minimum(sidx, N - 1)],
                    jnp.full((NLANES,), 1.0, jnp.float32),
                    mask=lanes < TOPK)

    pltpu.sync_copy(cnts_ref, out_hbm.at[wid])


_SC_CP = pltpu.CompilerParams()
if "needs_layout_passes" in pltpu.CompilerParams.__dataclass_fields__:
    _SC_CP = dataclasses.replace(_SC_CP, needs_layout_passes=False)


@functools.partial(
    pl.kernel,
    out_type=jax.ShapeDtypeStruct((NWORKERS, B * TOPK, N), jnp.float32),
    mesh=plsc.VectorSubcoreMesh(core_axis_name="core",
                                subcore_axis_name="subcore"),
    compiler_params=_SC_CP,
    scratch_types=[
        pltpu.VMEM((NLANES, N), jnp.int32),
        pltpu.VMEM((NLANES, CAP), jnp.int32),
        pltpu.VMEM((B * TOPK, N), jnp.float32),
    ],
)
def _sc_topk(keys_hbm, out_hbm, tile_ref, cand_ref, cnts_ref):
    _sc_topk_kernel(keys_hbm, out_hbm, tile_ref, cand_ref, cnts_ref)


def _select_kernel(parts_ref, x_ref, cls_ref, out_ref):
    counts = jnp.sum(parts_ref[:, 0], axis=0)
    sel = jnp.dot(counts, x_ref[0],
                  preferred_element_type=jnp.float32) * (1.0 / NUM_SAMPLES)
    out_ref[0, 0] = cls_ref[0, 0]
    out_ref[0, 1:] = sel


@jax.jit
def kernel(x, input_ids, attention_mask, ln_g, ln_b, W1, b1, W2, W3, noise):
    Bn, Nn, D = x.shape
    am_new = jnp.concatenate(
        [attention_mask[:, 1:], jnp.zeros((Bn, 1), attention_mask.dtype)],
        axis=1)
    ids3 = input_ids.reshape(Bn, 1, Nn)
    amn3 = am_new.reshape(Bn, 1, Nn)
    lng2 = ln_g.reshape(1, D)
    lnb2 = ln_b.reshape(1, D)
    b12 = b1.reshape(1, C)
    W2a = W2[:C]
    W2b = W2[C:]

    score, cls = pl.pallas_call(
        _scores_kernel,
        grid=(Bn,),
        in_specs=[
            pl.BlockSpec((1, Nn, D), lambda b: (b, 0, 0)),
            pl.BlockSpec((1, 1, Nn), lambda b: (b, 0, 0)),
            pl.BlockSpec((1, 1, Nn), lambda b: (b, 0, 0)),
            pl.BlockSpec((1, D), lambda b: (0, 0)),
            pl.BlockSpec((1, D), lambda b: (0, 0)),
            pl.BlockSpec((D, C), lambda b: (0, 0)),
            pl.BlockSpec((1, C), lambda b: (0, 0)),
            pl.BlockSpec((C, C), lambda b: (0, 0)),
            pl.BlockSpec((C, C), lambda b: (0, 0)),
            pl.BlockSpec((C, 1), lambda b: (0, 0)),
        ],
        out_specs=[
            pl.BlockSpec((1, 1, Nn), lambda b: (b, 0, 0)),
            pl.BlockSpec((1, 1, D), lambda b: (b, 0, 0)),
        ],
        out_shape=[
            jax.ShapeDtypeStruct((Bn, 1, Nn), jnp.float32),
            jax.ShapeDtypeStruct((Bn, 1, D), jnp.float32),
        ],
    )(x, ids3, amn3, lng2, lnb2, W1, b12, W2a, W2b, W3)

    keys = pl.pallas_call(
        _pack_kernel,
        grid=(Bn,),
        in_specs=[
            pl.BlockSpec((1, NUM_SAMPLES, Nn), lambda b: (b, 0, 0)),
            pl.BlockSpec((1, 1, Nn), lambda b: (b, 0, 0)),
        ],
        out_specs=pl.BlockSpec((1, NUM_SAMPLES, Nn), lambda b: (b, 0, 0)),
        out_shape=jax.ShapeDtypeStruct((Bn, NUM_SAMPLES, Nn), jnp.int32),
    )(noise, score)

    partials = _sc_topk(keys.reshape(NROWS, Nn))
    parts4 = partials.reshape(NWORKERS, Bn, TOPK, Nn)

    out = pl.pallas_call(
        _select_kernel,
        grid=(Bn,),
        in_specs=[
            pl.BlockSpec((NWORKERS, 1, TOPK, Nn), lambda b: (0, b, 0, 0)),
            pl.BlockSpec((1, Nn, D), lambda b: (b, 0, 0)),
            pl.BlockSpec((1, 1, D), lambda b: (b, 0, 0)),
        ],
        out_specs=pl.BlockSpec((1, 1 + TOPK, D), lambda b: (b, 0, 0)),
        out_shape=jax.ShapeDtypeStruct((Bn, 1 + TOPK, D), jnp.float32),
    )(parts4, x, cls)
    return out


# speedup vs baseline: 1.0715x; 1.0715x over previous
import dataclasses
import functools

import jax
import jax.numpy as jnp
from jax import lax
from jax.experimental import pallas as pl
from jax.experimental.pallas import tpu as pltpu
from jax.experimental.pallas import tpu_sc as plsc

EMBED_DIM = 768
TOPK = 8
NUM_SAMPLES = 500
SIGMA = 0.05
B, N = 4, 2048
C = EMBED_DIM // 2

NLANES = 16
NWORKERS = 32
SLOTS = NWORKERS // B
GROUPS_PER_TEC = 4
CAP = 256
NROWS = B * NUM_SAMPLES
NGROUPS = NROWS // NLANES

_INT_MIN = jnp.iinfo(jnp.int32).min
_INV_SQRT2 = 0.7071067811865476


def _gelu_exact(v):
    return v * 0.5 * (1.0 + jax.lax.erf(v * _INV_SQRT2))


def _scores_kernel(x_ref, ids_ref, amn_ref, lng_ref, lnb_ref,
                   w1_ref, b1_ref, w2a_ref, w2b_ref, w3_ref,
                   score_ref, cls_ref):
    xb = x_ref[0]
    mu = jnp.mean(xb, axis=-1, keepdims=True)
    xc = xb - mu
    var = jnp.mean(xc * xc, axis=-1, keepdims=True)
    ln = xc / jnp.sqrt(var + 1e-5) * lng_ref[0][None, :] + lnb_ref[0][None, :]
    h = _gelu_exact(
        jnp.dot(ln, w1_ref[...], preferred_element_type=jnp.float32)
        + b1_ref[0][None, :])

    ids = ids_ref[0]
    iota = jax.lax.broadcasted_iota(jnp.int32, (1, N), 1)
    ikey = ids * N + (N - 1 - iota)
    gmax = jnp.max(ikey)
    gsel = (ikey == gmax).astype(jnp.float32)
    h_g = jnp.dot(gsel, h, preferred_element_type=jnp.float32)
    cls_ref[0] = jnp.dot(gsel, xb, preferred_element_type=jnp.float32)

    bias2 = jnp.dot(h_g, w2b_ref[...], preferred_element_type=jnp.float32)
    o = _gelu_exact(
        jnp.dot(h, w2a_ref[...], preferred_element_type=jnp.float32) + bias2)
    s = jax.nn.sigmoid(
        jnp.dot(o, w3_ref[...], preferred_element_type=jnp.float32))
    score_ref[0] = s.reshape(1, N) * amn_ref[0]


def _pack_kernel(noise_ref, score_ref, keys_ref):
    iota = jax.lax.broadcasted_iota(jnp.int32, (1, N), 1)
    p = score_ref[0] + noise_ref[0] * SIGMA
    q = ((p + 0.5) * 262144.0).astype(jnp.int32)
    keys_ref[0] = q * N + (N - 1 - iota)


def _sc_topk_kernel(keys_hbm, out_hbm, tile_ref, cand_ref, cnts_ref):
    lanes = lax.iota(jnp.int32, NLANES)
    zero16f = jnp.zeros((NLANES,), jnp.float32)
    wid = lax.axis_index("subcore") * 2 + lax.axis_index("core")

    @pl.loop(0, B * TOPK * N, step=8 * NLANES)
    def _(cc):
        for u in range(8):
            cnts_ref[pl.ds(cc + u * NLANES, NLANES)] = zero16f

    lane_row = lanes * N
    lane_cap = lanes * CAP

    for t in range(GROUPS_PER_TEC):
        g = wid + t * NWORKERS

        @pl.when(g < NGROUPS)
        def _():
            gbase = NLANES * g
            pltpu.sync_copy(keys_hbm.at[pl.ds(gbase * N, NLANES * N)],
                            tile_ref)

            def p1_body(i, carry):
                flat = carry[0]
                ms = list(carry[1:])
                for u in range(16):
                    v = plsc.load_gather(tile_ref, [flat + u])
                    ms[u & 7] = jnp.maximum(ms[u & 7], v)
                return (flat + 16,) + tuple(ms)

            minit = tuple(jnp.full((NLANES,), _INT_MIN, jnp.int32)
                          for _ in range(8))
            res = lax.fori_loop(0, N // 16, p1_body, (lane_row,) + minit)
            ms = res[1:]
            thr = ms[0]
            for u in range(1, 8):
                thr = jnp.minimum(thr, ms[u])

            def p2_body(i, carry):
                flat, cnt = carry
                for u in range(8):
                    v = plsc.load_gather(tile_ref, [flat + u])
                    msk = v >= thr
                    addr = lane_cap + jnp.minimum(cnt, CAP - 1)
                    plsc.store_scatter(cand_ref, [addr], v, mask=msk)
                    cnt = cnt + msk.astype(jnp.int32)
                return (flat + 8, cnt)

            _unused, cnt = lax.fori_loop(
                0, N // 8, p2_body,
                (lane_row, jnp.zeros((NLANES,), jnp.int32)))

            for l in range(NLANES):
                cl = jnp.max(jnp.where(lanes == l, cnt, 0))
                nv = (cl + NLANES - 1) // NLANES
                bl = (gbase + l) // NUM_SAMPLES

                def merge(vi, best):
                    vec = cand_ref[pl.ds(l * CAP + vi * NLANES, NLANES)]
                    valid = (vi * NLANES + lanes) < cl
                    vec = jnp.where(valid, vec, lanes)
                    asc = lax.sort(vec)
                    m = jnp.maximum(best, asc)
                    best, _u2 = plsc.sort_key_val(m, m, descending=True)
                    return best

                best = lax.fori_loop(0, nv, merge, 15 - lanes)
                idx = (N - 1) - (best & (N - 1))
                svals = jnp.where(lanes < TOPK, idx, 4064 + lanes)
                sidx = lax.sort(svals)
                plsc.addupdate_scatter(
                    cnts_ref,
                    [(bl * TOPK + jnp.minimum(lanes, TOPK - 1)) * N
                     + jnp.minimum(sidx, N - 1)],
                    jnp.full((NLANES,), 1.0, jnp.float32),
                    mask=lanes < TOPK)

    pltpu.sync_copy(cnts_ref, out_hbm.at[wid])


_SC_CP = pltpu.CompilerParams()
if "needs_layout_passes" in pltpu.CompilerParams.__dataclass_fields__:
    _SC_CP = dataclasses.replace(_SC_CP, needs_layout_passes=False)


@functools.partial(
    pl.kernel,
    out_type=jax.ShapeDtypeStruct((NWORKERS, B * TOPK * N), jnp.float32),
    mesh=plsc.VectorSubcoreMesh(core_axis_name="core",
                                subcore_axis_name="subcore"),
    compiler_params=_SC_CP,
    scratch_types=[
        pltpu.VMEM((NLANES * N,), jnp.int32),
        pltpu.VMEM((NLANES * CAP,), jnp.int32),
        pltpu.VMEM((B * TOPK * N,), jnp.float32),
    ],
)
def _sc_topk(keys_hbm, out_hbm, tile_ref, cand_ref, cnts_ref):
    _sc_topk_kernel(keys_hbm, out_hbm, tile_ref, cand_ref, cnts_ref)


def _select_kernel(parts_ref, x_ref, cls_ref, out_ref):
    counts = jnp.sum(parts_ref[:, 0], axis=0)
    sel = jnp.dot(counts, x_ref[0],
                  preferred_element_type=jnp.float32) * (1.0 / NUM_SAMPLES)
    out_ref[0, 0] = cls_ref[0, 0]
    out_ref[0, 1:] = sel


@jax.jit
def kernel(x, input_ids, attention_mask, ln_g, ln_b, W1, b1, W2, W3, noise):
    Bn, Nn, D = x.shape
    am_new = jnp.concatenate(
        [attention_mask[:, 1:], jnp.zeros((Bn, 1), attention_mask.dtype)],
        axis=1)
    ids3 = input_ids.reshape(Bn, 1, Nn)
    amn3 = am_new.reshape(Bn, 1, Nn)
    lng2 = ln_g.reshape(1, D)
    lnb2 = ln_b.reshape(1, D)
    b12 = b1.reshape(1, C)
    W2a = W2[:C]
    W2b = W2[C:]

    score, cls = pl.pallas_call(
        _scores_kernel,
        grid=(Bn,),
        in_specs=[
            pl.BlockSpec((1, Nn, D), lambda b: (b, 0, 0)),
            pl.BlockSpec((1, 1, Nn), lambda b: (b, 0, 0)),
            pl.BlockSpec((1, 1, Nn), lambda b: (b, 0, 0)),
            pl.BlockSpec((1, D), lambda b: (0, 0)),
            pl.BlockSpec((1, D), lambda b: (0, 0)),
            pl.BlockSpec((D, C), lambda b: (0, 0)),
            pl.BlockSpec((1, C), lambda b: (0, 0)),
            pl.BlockSpec((C, C), lambda b: (0, 0)),
            pl.BlockSpec((C, C), lambda b: (0, 0)),
            pl.BlockSpec((C, 1), lambda b: (0, 0)),
        ],
        out_specs=[
            pl.BlockSpec((1, 1, Nn), lambda b: (b, 0, 0)),
            pl.BlockSpec((1, 1, D), lambda b: (b, 0, 0)),
        ],
        out_shape=[
            jax.ShapeDtypeStruct((Bn, 1, Nn), jnp.float32),
            jax.ShapeDtypeStruct((Bn, 1, D), jnp.float32),
        ],
    )(x, ids3, amn3, lng2, lnb2, W1, b12, W2a, W2b, W3)

    keys = pl.pallas_call(
        _pack_kernel,
        grid=(Bn,),
        in_specs=[
            pl.BlockSpec((1, NUM_SAMPLES, Nn), lambda b: (b, 0, 0)),
            pl.BlockSpec((1, 1, Nn), lambda b: (b, 0, 0)),
        ],
        out_specs=pl.BlockSpec((1, NUM_SAMPLES, Nn), lambda b: (b, 0, 0)),
        out_shape=jax.ShapeDtypeStruct((Bn, NUM_SAMPLES, Nn), jnp.int32),
    )(noise, score)

    partials = _sc_topk(keys.reshape(NROWS * Nn))
    parts4 = partials.reshape(NWORKERS, Bn, TOPK, Nn)

    out = pl.pallas_call(
        _select_kernel,
        grid=(Bn,),
        in_specs=[
            pl.BlockSpec((NWORKERS, 1, TOPK, Nn), lambda b: (0, b, 0, 0)),
            pl.BlockSpec((1, Nn, D), lambda b: (b, 0, 0)),
            pl.BlockSpec((1, 1, D), lambda b: (b, 0, 0)),
        ],
        out_specs=pl.BlockSpec((1, 1 + TOPK, D), lambda b: (b, 0, 0)),
        out_shape=jax.ShapeDtypeStruct((Bn, 1 + TOPK, D), jnp.float32),
    )(parts4, x, cls)
    return out
